# baseline (speedup 1.0000x reference)
import jax
import jax.numpy as jnp
from jax import lax
from jax.experimental import pallas as pl
from jax.experimental.pallas import tpu as pltpu
from jax.experimental.pallas import tpu_sc as plsc

N_NODES = 100000
N_SPECIES = 119
EMBED_DIM = 256
NC = 2
NS = 16
NW = NC * NS

LANES = 16
CHUNK = 16
IDX_OFF = 8

ROWS_MAIN = 3136
ROWS_LAST = N_NODES - 31 * ROWS_MAIN
N_CHUNKS = ROWS_MAIN // CHUNK


NBUF = 2


def _gather_body(idx_hbm, table_hbm, out_hbm,
                 table_v, idx0, idx1,
                 tsem, isem0, isem1, osem0, osem1):
    wid = lax.axis_index("s") * NC + lax.axis_index("c")
    base = wid * ROWS_MAIN
    count = jnp.where(wid == NW - 1, ROWS_LAST, ROWS_MAIN)
    last_start = base + count - CHUNK

    idx_bufs = (idx0, idx1)
    isems = (isem0, isem1)
    osems = (osem0, osem1)

    def cstart(j):
        return jnp.minimum(base + j * CHUNK, last_start)

    def idx_copy(j, b):
        return pltpu.make_async_copy(
            idx_hbm.at[pl.ds(cstart(j), CHUNK)],
            idx_bufs[b].at[pl.ds(IDX_OFF, CHUNK)], isems[b])

    def row_copy(src_row, dst_row, b):
        return pltpu.make_async_copy(
            table_v.at[pl.ds(src_row * EMBED_DIM, EMBED_DIM)],
            out_hbm.at[pl.ds(dst_row * EMBED_DIM, EMBED_DIM)], osems[b])

    def issue(j, b):
        iv = idx_bufs[b][pl.ds(IDX_OFF, CHUNK)]
        start = cstart(j)
        for r in range(CHUNK):
            row_copy(iv[r], start + r, b).start()

    def drain(b):
        for _ in range(CHUNK):
            row_copy(0, 0, b).wait()

    tcp = pltpu.make_async_copy(table_hbm, table_v, tsem)
    tcp.start()
    for b in range(NBUF):
        idx_copy(b, b).start()
    tcp.wait()

    for b in range(NBUF):
        idx_copy(b, b).wait()
        issue(b, b)
        idx_copy(b + NBUF, b).start()

    def body(p, carry):
        for b in range(NBUF):
            j = NBUF * p + b
            idx_copy(j, b).wait()
            drain(b)
            issue(j, b)
            idx_copy(j + NBUF, b).start()
        return carry

    lax.fori_loop(1, N_CHUNKS // NBUF, body, None)

    for b in range(NBUF):
        idx_copy(N_CHUNKS + b, b).wait()
        drain(b)


@jax.jit
def _gather(node_specie, embeddings_flat):
    mesh = plsc.VectorSubcoreMesh(
        core_axis_name="c", subcore_axis_name="s",
        num_cores=NC, num_subcores=NS)
    out_flat = pl.kernel(
        _gather_body,
        out_type=jax.ShapeDtypeStruct((N_NODES * EMBED_DIM,), jnp.float32),
        mesh=mesh,
        compiler_params=pltpu.CompilerParams(needs_layout_passes=False),
        scratch_types=(
            [pltpu.VMEM((N_SPECIES * EMBED_DIM,), jnp.float32)]
            + [pltpu.VMEM((IDX_OFF + CHUNK,), jnp.int32)] * NBUF
            + [pltpu.SemaphoreType.DMA] * (1 + 2 * NBUF)
        ),
        name="embedding_gather_sc",
    )(node_specie, embeddings_flat)
    return out_flat.reshape(N_NODES, EMBED_DIM)


def kernel(node_specie, embeddings):
    return _gather(node_specie.astype(jnp.int32),
                   embeddings.reshape(N_SPECIES * EMBED_DIM))
